# Initial kernel scaffold; baseline (speedup 1.0000x reference)
import jax
import jax.numpy as jnp
from jax.experimental import pallas as pl


def kernel(input, target):
    raise NotImplementedError("write your pallas kernel here")


# speedup vs baseline: 1.3058x; 1.3058x over previous
import functools

import jax
import jax.numpy as jnp
from jax import lax
from jax.experimental import pallas as pl
from jax.experimental.pallas import tpu as pltpu
from jax.experimental.pallas import tpu_sc as plsc

_B = 1024
_V = 100000
_ROWS_PER_STEP = 16


def _gather_partials_sc(x, t):
    info = plsc.get_sparse_core_info()
    nc, ns = info.num_cores, info.num_subcores
    nw = nc * ns
    rpw = _B // nw
    mesh = plsc.VectorSubcoreMesh(core_axis_name="c", subcore_axis_name="s")

    @functools.partial(
        pl.kernel,
        mesh=mesh,
        out_type=jax.ShapeDtypeStruct((nw, 16), jnp.float32),
        compiler_params=pltpu.CompilerParams(needs_layout_passes=False),
        scratch_types=[
            pltpu.VMEM((rpw,), jnp.int32),
            pltpu.VMEM((8, 8, 128), jnp.float32),
            pltpu.VMEM((16,), jnp.float32),
            pltpu.SemaphoreType.DMA,
        ],
    )
    def gather_kernel(x_hbm, t_hbm, out_hbm, t_v, tile_v, acc_v, sem):
        wid = lax.axis_index("s") * nc + lax.axis_index("c")
        base = wid * rpw
        pltpu.sync_copy(t_hbm.at[pl.ds(base, rpw)], t_v)
        lane = lax.broadcasted_iota(jnp.int32, (16,), 0)
        acc = jnp.zeros((16,), jnp.float32)
        chunk = 8
        for c in range(rpw // chunk):
            tv = t_v[pl.ds((c // 2) * 16, 16)]
            copies = []
            cols = []
            for j in range(chunk):
                r = c * chunk + j
                t_s = jnp.sum(jnp.where(lane == (r % 16), tv, 0))
                t0 = pl.multiple_of((t_s >> 7) << 7, 128)
                cols.append(t_s & 127)
                copies.append(
                    pltpu.async_copy(
                        x_hbm.at[pl.ds(base + c * chunk, 8), pl.ds(t0, 128)],
                        tile_v.at[j],
                        sem,
                    )
                )
            for cp in copies:
                cp.wait()
            for j in range(chunk):
                col = cols[j]
                seg, ln = col >> 4, col & 15
                for k in range(8):
                    v = tile_v[j, j, pl.ds(k * 16, 16)]
                    acc = acc + jnp.where((seg == k) & (lane == ln), v, 0.0)
        acc_v[...] = acc
        pltpu.sync_copy(acc_v, out_hbm.at[wid])

    return gather_kernel(x, t)


def _loss_tc(x, gpart, nw):
    grid = _B // _ROWS_PER_STEP

    def body(x_ref, g_ref, o_ref):
        i = pl.program_id(0)
        xv = x_ref[...]
        s = jnp.sum(jnp.maximum(xv, 0.0) + jnp.log1p(jnp.exp(-jnp.abs(xv))))

        @pl.when(i == 0)
        def _():
            o_ref[0, 0] = s - jnp.sum(g_ref[...])

        @pl.when(i > 0)
        def _():
            o_ref[0, 0] = o_ref[0, 0] + s

        @pl.when(i == grid - 1)
        def _():
            o_ref[0, 0] = o_ref[0, 0] * (1.0 / (_B * _V))

    return pl.pallas_call(
        body,
        grid=(grid,),
        in_specs=[
            pl.BlockSpec((_ROWS_PER_STEP, _V), lambda i: (i, 0)),
            pl.BlockSpec((nw, 16), lambda i: (0, 0)),
        ],
        out_specs=pl.BlockSpec(memory_space=pltpu.SMEM),
        out_shape=jax.ShapeDtypeStruct((1, 1), jnp.float32),
    )(x, gpart)


def kernel(input, target):
    t = target.astype(jnp.int32)
    gpart = _gather_partials_sc(input, t)
    out = _loss_tc(input, gpart, gpart.shape[0])
    return out[0, 0]


# speedup vs baseline: 1.6688x; 1.2780x over previous
import functools

import jax
import jax.numpy as jnp
from jax import lax
from jax.experimental import pallas as pl
from jax.experimental.pallas import tpu as pltpu
from jax.experimental.pallas import tpu_sc as plsc

_B = 1024
_V = 100000
_ROWS_PER_STEP = 16


def _gather_partials_sc(x, t):
    info = plsc.get_sparse_core_info()
    nc, ns = info.num_cores, info.num_subcores
    nw = nc * ns
    rpw = _B // nw
    mesh = plsc.VectorSubcoreMesh(core_axis_name="c", subcore_axis_name="s")

    @functools.partial(
        pl.kernel,
        mesh=mesh,
        out_type=jax.ShapeDtypeStruct((nw, 16), jnp.float32),
        compiler_params=pltpu.CompilerParams(needs_layout_passes=False),
        scratch_types=[
            pltpu.VMEM((rpw,), jnp.int32),
            pltpu.VMEM((8, 8, 128), jnp.float32),
            pltpu.VMEM((16,), jnp.float32),
            pltpu.SemaphoreType.DMA,
        ],
    )
    def gather_kernel(x_hbm, t_hbm, out_hbm, t_v, tile_v, acc_v, sem):
        wid = lax.axis_index("s") * nc + lax.axis_index("c")
        base = wid * rpw
        pltpu.sync_copy(t_hbm.at[pl.ds(base, rpw)], t_v)
        lane = lax.broadcasted_iota(jnp.int32, (16,), 0)
        acc = jnp.zeros((16,), jnp.float32)
        chunk = 8
        for c in range(rpw // chunk):
            tv = t_v[pl.ds((c // 2) * 16, 16)]
            copies = []
            cols = []
            for j in range(chunk):
                r = c * chunk + j
                t_s = jnp.sum(jnp.where(lane == (r % 16), tv, 0))
                t0 = pl.multiple_of((t_s >> 7) << 7, 128)
                cols.append(t_s & 127)
                copies.append(
                    pltpu.async_copy(
                        x_hbm.at[pl.ds(base + c * chunk, 8), pl.ds(t0, 128)],
                        tile_v.at[j],
                        sem,
                    )
                )
            for cp in copies:
                cp.wait()
            for j in range(chunk):
                col = cols[j]
                seg, ln = col >> 4, col & 15
                for k in range(8):
                    v = tile_v[j, j, pl.ds(k * 16, 16)]
                    acc = acc + jnp.where((seg == k) & (lane == ln), v, 0.0)
        acc_v[...] = acc
        pltpu.sync_copy(acc_v, out_hbm.at[wid])

    return gather_kernel(x, t)


def _loss_tc(x, gpart, nw):
    grid = _B // _ROWS_PER_STEP

    neg_log2e = -1.4426950408889634
    ln2 = 0.6931471805599453
    nstreams = 2
    nsteps = grid // nstreams

    cuts = [(k * 12544, 12544) for k in range(7)] + [(87808, 12192)]

    def one(xv):
        s_rl = 0.0
        s_lg = 0.0
        for c0, w in cuts:
            xs = xv[:, c0:c0 + w]
            e = jnp.exp2(jnp.abs(xs) * neg_log2e)
            s_lg = s_lg + jnp.sum(jnp.log2(1.0 + e))
            s_rl = s_rl + jnp.sum(jnp.maximum(xs, 0.0))
        return s_rl + ln2 * s_lg

    def body(xa_ref, xb_ref, g_ref, o_ref):
        i = pl.program_id(0)
        s = one(xa_ref[...]) + one(xb_ref[...])

        @pl.when(i == 0)
        def _():
            o_ref[0, 0] = s - jnp.sum(g_ref[...])

        @pl.when(i > 0)
        def _():
            o_ref[0, 0] = o_ref[0, 0] + s

        @pl.when(i == nsteps - 1)
        def _():
            o_ref[0, 0] = o_ref[0, 0] * (1.0 / (_B * _V))

    def mkspec(k):
        return pl.BlockSpec((_ROWS_PER_STEP, _V), lambda i, k=k: (i + k * nsteps, 0))

    return pl.pallas_call(
        body,
        grid=(nsteps,),
        in_specs=[mkspec(k) for k in range(nstreams)] + [
            pl.BlockSpec((nw, 16), lambda i: (0, 0)),
        ],
        out_specs=pl.BlockSpec(memory_space=pltpu.SMEM),
        out_shape=jax.ShapeDtypeStruct((1, 1), jnp.float32),
        compiler_params=pltpu.CompilerParams(vmem_limit_bytes=117 * 1024 * 1024),
    )(*([x] * nstreams + [gpart]))


def kernel(input, target):
    t = target.astype(jnp.int32)
    gpart = _gather_partials_sc(input, t)
    out = _loss_tc(input, gpart, gpart.shape[0])
    return out[0, 0]


# speedup vs baseline: 1.6880x; 1.0115x over previous
import functools

import jax
import jax.numpy as jnp
from jax import lax
from jax.experimental import pallas as pl
from jax.experimental.pallas import tpu as pltpu
from jax.experimental.pallas import tpu_sc as plsc

_B = 1024
_V = 100000
_ROWS_PER_STEP = 16


def _gather_partials_sc(x, t):
    info = plsc.get_sparse_core_info()
    nc, ns = info.num_cores, info.num_subcores
    nw = nc * ns
    rpw = _B // nw
    mesh = plsc.VectorSubcoreMesh(core_axis_name="c", subcore_axis_name="s")

    @functools.partial(
        pl.kernel,
        mesh=mesh,
        out_type=jax.ShapeDtypeStruct((nw, 16), jnp.float32),
        compiler_params=pltpu.CompilerParams(needs_layout_passes=False),
        scratch_types=[
            pltpu.VMEM((rpw,), jnp.int32),
            pltpu.VMEM((8, 8, 128), jnp.float32),
            pltpu.VMEM((16,), jnp.float32),
            pltpu.SemaphoreType.DMA,
        ],
    )
    def gather_kernel(x_hbm, t_hbm, out_hbm, t_v, tile_v, acc_v, sem):
        wid = lax.axis_index("s") * nc + lax.axis_index("c")
        base = wid * rpw
        pltpu.sync_copy(t_hbm.at[pl.ds(base, rpw)], t_v)
        lane = lax.broadcasted_iota(jnp.int32, (16,), 0)
        acc = jnp.zeros((16,), jnp.float32)
        chunk = 8
        for c in range(rpw // chunk):
            tv = t_v[pl.ds((c // 2) * 16, 16)]
            copies = []
            cols = []
            for j in range(chunk):
                r = c * chunk + j
                t_s = jnp.sum(jnp.where(lane == (r % 16), tv, 0))
                t0 = pl.multiple_of((t_s >> 7) << 7, 128)
                cols.append(t_s & 127)
                copies.append(
                    pltpu.async_copy(
                        x_hbm.at[pl.ds(base + c * chunk, 8), pl.ds(t0, 128)],
                        tile_v.at[j],
                        sem,
                    )
                )
            for cp in copies:
                cp.wait()
            for j in range(chunk):
                col = cols[j]
                seg, ln = col >> 4, col & 15
                for k in range(8):
                    v = tile_v[j, j, pl.ds(k * 16, 16)]
                    acc = acc + jnp.where((seg == k) & (lane == ln), v, 0.0)
        acc_v[...] = acc
        pltpu.sync_copy(acc_v, out_hbm.at[wid])

    return gather_kernel(x, t)


def _loss_tc(x):
    grid = _B // _ROWS_PER_STEP

    neg_log2e = -1.4426950408889634
    ln2 = 0.6931471805599453
    nstreams = 2
    nsteps = grid // nstreams

    cuts = [(k * 12544, 12544) for k in range(7)] + [(87808, 12192)]

    def one(xv):
        s_rl = 0.0
        s_lg = 0.0
        for c0, w in cuts:
            xs = xv[:, c0:c0 + w]
            e = jnp.exp2(jnp.abs(xs) * neg_log2e)
            s_lg = s_lg + jnp.sum(jnp.log2(1.0 + e))
            s_rl = s_rl + jnp.sum(jnp.maximum(xs, 0.0))
        return s_rl + ln2 * s_lg

    def body(xa_ref, xb_ref, o_ref):
        i = pl.program_id(0)
        s = one(xa_ref[...]) + one(xb_ref[...])

        @pl.when(i == 0)
        def _():
            o_ref[0, 0] = s

        @pl.when(i > 0)
        def _():
            o_ref[0, 0] = o_ref[0, 0] + s

    def mkspec(k):
        return pl.BlockSpec((_ROWS_PER_STEP, _V), lambda i, k=k: (i + k * nsteps, 0))

    return pl.pallas_call(
        body,
        grid=(nsteps,),
        in_specs=[mkspec(k) for k in range(nstreams)],
        out_specs=pl.BlockSpec(memory_space=pltpu.SMEM),
        out_shape=jax.ShapeDtypeStruct((1, 1), jnp.float32),
    )(*([x] * nstreams))


def kernel(input, target):
    t = target.astype(jnp.int32)
    gpart = _gather_partials_sc(input, t)
    dense = _loss_tc(input)
    return (dense[0, 0] - jnp.sum(gpart)) * (1.0 / (_B * _V))
